# Initial kernel scaffold; baseline (speedup 1.0000x reference)
import jax
import jax.numpy as jnp
from jax.experimental import pallas as pl


def kernel(p, x, W1, g1, b1, W2, g2, b2, Wskip, bskip):
    raise NotImplementedError("write your pallas kernel here")


# speedup vs baseline: 21.0941x; 21.0941x over previous
import functools

import jax
import jax.numpy as jnp
from jax import lax
from jax.experimental import pallas as pl
from jax.experimental.pallas import tpu as pltpu
from jax.experimental.pallas import tpu_sc as plsc

_B = 4
_N = 4096
_CIN = 128
_COUT = 256
_M = 1024
_K = 32
_RADIUS = 0.1
_CMID = 128
_EPS = 1e-5
_S = _B * _M * _K
_BM = _B * _M
_CNT = float(_S)


_FC = 128


def _fps_body(px_ref, py_ref, pz_ref, idx_ref, nx_ref, ny_ref, nz_ref):
    px = px_ref[...]
    py = py_ref[...]
    pz = pz_ref[...]
    iota = lax.broadcasted_iota(jnp.int32, (_B, _N), 1)
    boff = lax.broadcasted_iota(jnp.int32, (_B, 1), 0) * _N
    lane = lax.broadcasted_iota(jnp.int32, (_B, _FC), 1)

    def body(j, carry):
        dists, cur, bi, bx, by, bz = carry
        onehot = iota == cur
        lx = jnp.sum(jnp.where(onehot, px, 0.0), axis=1, keepdims=True)
        ly = jnp.sum(jnp.where(onehot, py, 0.0), axis=1, keepdims=True)
        lz = jnp.sum(jnp.where(onehot, pz, 0.0), axis=1, keepdims=True)
        sel = lane == j
        bi = jnp.where(sel, cur + boff, bi)
        bx = jnp.where(sel, lx, bx)
        by = jnp.where(sel, ly, by)
        bz = jnp.where(sel, lz, bz)
        dx = px - lx
        dy = py - ly
        dz = pz - lz
        d = (dx * dx + dy * dy) + dz * dz
        dists = jnp.minimum(dists, d)
        maxv = jnp.max(dists, axis=1, keepdims=True)
        nxt = jnp.min(jnp.where(dists == maxv, iota, _N), axis=1, keepdims=True)
        return dists, nxt, bi, bx, by, bz

    dists = jnp.full((_B, _N), 1e10, jnp.float32)
    cur = jnp.zeros((_B, 1), jnp.int32)
    for c in range(_M // _FC):
        bi = jnp.zeros((_B, _FC), jnp.int32)
        bx = jnp.zeros((_B, _FC), jnp.float32)
        by = jnp.zeros((_B, _FC), jnp.float32)
        bz = jnp.zeros((_B, _FC), jnp.float32)
        dists, cur, bi, bx, by, bz = lax.fori_loop(
            0, _FC, body, (dists, cur, bi, bx, by, bz))
        idx_ref[:, pl.ds(c * _FC, _FC)] = bi
        nx_ref[:, pl.ds(c * _FC, _FC)] = bx
        ny_ref[:, pl.ds(c * _FC, _FC)] = by
        nz_ref[:, pl.ds(c * _FC, _FC)] = bz


def _fps(px, py, pz):
    return pl.pallas_call(
        _fps_body,
        out_shape=(
            jax.ShapeDtypeStruct((_B, _M), jnp.int32),
            jax.ShapeDtypeStruct((_B, _M), jnp.float32),
            jax.ShapeDtypeStruct((_B, _M), jnp.float32),
            jax.ShapeDtypeStruct((_B, _M), jnp.float32),
        ),
    )(px, py, pz)


_MB = 256


def _bq_body(px_ref, py_ref, pz_ref, nx_ref, ny_ref, nz_ref, nbr_ref):
    b = pl.program_id(0)
    px = jnp.reshape(px_ref[...], (1, _N))
    py = jnp.reshape(py_ref[...], (1, _N))
    pz = jnp.reshape(pz_ref[...], (1, _N))
    nx = jnp.reshape(nx_ref[...], (_MB, 1))
    ny = jnp.reshape(ny_ref[...], (_MB, 1))
    nz = jnp.reshape(nz_ref[...], (_MB, 1))
    sn = (px * px + py * py) + pz * pz
    qn = (nx * nx + ny * ny) + nz * nz
    pxb = px.astype(jnp.bfloat16).astype(jnp.float32)
    pyb = py.astype(jnp.bfloat16).astype(jnp.float32)
    pzb = pz.astype(jnp.bfloat16).astype(jnp.float32)
    nxb = nx.astype(jnp.bfloat16).astype(jnp.float32)
    nyb = ny.astype(jnp.bfloat16).astype(jnp.float32)
    nzb = nz.astype(jnp.bfloat16).astype(jnp.float32)
    cross = (nxb * pxb + nyb * pyb) + nzb * pzb
    d2 = jnp.maximum((qn + sn) - 2.0 * cross, 0.0)
    fN = jnp.float32(_N)
    iota = lax.broadcasted_iota(jnp.int32, (_MB, _N), 1).astype(jnp.float32)
    cand = jnp.where(d2 < _RADIUS**2, iota, fN)
    cols = []
    for _ in range(_K):
        cur = jnp.min(cand, axis=1, keepdims=True)
        cols.append(cur)
        cand = jnp.where(cand == cur, fN, cand)
    nbrf = jnp.concatenate(cols, axis=1)
    first = nbrf[:, 0:1]
    nbrf = jnp.where(nbrf < (_N - 0.5), nbrf, first)
    nbr = nbrf.astype(jnp.int32) + b * _N
    nbr_ref[...] = jnp.reshape(nbr, (1, _MB, _K))


def _ball_query(px, py, pz, nx, ny, nz):
    return pl.pallas_call(
        _bq_body,
        grid=(_B, _M // _MB),
        in_specs=[
            pl.BlockSpec((_N,), lambda b, j: (b,)),
            pl.BlockSpec((_N,), lambda b, j: (b,)),
            pl.BlockSpec((_N,), lambda b, j: (b,)),
            pl.BlockSpec((_MB,), lambda b, j: (b * (_M // _MB) + j,)),
            pl.BlockSpec((_MB,), lambda b, j: (b * (_M // _MB) + j,)),
            pl.BlockSpec((_MB,), lambda b, j: (b * (_M // _MB) + j,)),
        ],
        out_specs=pl.BlockSpec((1, _MB, _K), lambda b, j: (b, j, 0)),
        out_shape=jax.ShapeDtypeStruct((_B, _M, _K), jnp.int32),
    )(jnp.reshape(px, (_B * _N,)), jnp.reshape(py, (_B * _N,)),
      jnp.reshape(pz, (_B * _N,)), jnp.reshape(nx, (_BM,)),
      jnp.reshape(ny, (_BM,)), jnp.reshape(nz, (_BM,)))


_CH = 256
_GD = 16


def _pw_body(pp_ref, w1p_ref, pw_ref):
    pw_ref[...] = jnp.dot(pp_ref[...], w1p_ref[...],
                          preferred_element_type=jnp.float32)


def _pw(ppad, w1p_t):
    return pl.pallas_call(
        _pw_body,
        grid=(8,),
        in_specs=[
            pl.BlockSpec((_B * _N // 8, _GD), lambda g: (g, 0)),
            pl.BlockSpec((_GD, _CMID), lambda g: (0, 0)),
        ],
        out_specs=pl.BlockSpec((_B * _N // 8, _CMID), lambda g: (g, 0)),
        out_shape=jax.ShapeDtypeStruct((_B * _N, _CMID), jnp.float32),
    )(ppad, w1p_t)


def _sc_gather(xt, pw, nbr_flat, idx_flat):
    info = plsc.get_sparse_core_info()
    nc, ns = info.num_cores, info.num_subcores
    nw = nc * ns
    per_w = _S // nw
    nch = per_w // _CH
    ib = _BM // nw
    mesh = plsc.VectorSubcoreMesh(core_axis_name="c", subcore_axis_name="s")

    @functools.partial(
        pl.kernel,
        mesh=mesh,
        out_type=(
            jax.ShapeDtypeStruct((_S, _CIN), jnp.float32),
            jax.ShapeDtypeStruct((_S, _CMID), jnp.float32),
            jax.ShapeDtypeStruct((_BM, _CIN), jnp.float32),
        ),
        scratch_types=[
            pltpu.VMEM((_CH,), jnp.int32),
            pltpu.VMEM((_CH, _CIN), jnp.float32),
            pltpu.VMEM((_CH, _CMID), jnp.float32),
            pltpu.VMEM((ib,), jnp.int32),
            pltpu.VMEM((ib, _CIN), jnp.float32),
            pltpu.SemaphoreType.DMA,
        ],
    )
    def k(xt_hbm, pw_hbm, nbr_hbm, idx_hbm, xj_out, pwj_out, id_out,
          idxv, rowsv, pwv, iidxv, irowsv, sem):
        wid = lax.axis_index("s") * nc + lax.axis_index("c")
        base = wid * per_w
        for t in range(nch):
            off = base + t * _CH
            pltpu.sync_copy(nbr_hbm.at[pl.ds(off, _CH)], idxv)
            pltpu.async_copy(xt_hbm.at[idxv], rowsv, sem).wait()
            pltpu.sync_copy(rowsv, xj_out.at[pl.ds(off, _CH)])
            pltpu.async_copy(pw_hbm.at[idxv], pwv, sem).wait()
            pltpu.sync_copy(pwv, pwj_out.at[pl.ds(off, _CH)])
        ibase = wid * ib
        pltpu.sync_copy(idx_hbm.at[pl.ds(ibase, ib)], iidxv)
        pltpu.async_copy(xt_hbm.at[iidxv], irowsv, sem).wait()
        pltpu.sync_copy(irowsv, id_out.at[pl.ds(ibase, ib)])

    return k(xt, pw, nbr_flat, idx_flat)


_G = 128
_NG = _BM // _G


def _c1_body(xj_ref, pwj_ref, np_ref, w1x_ref, w1p_ref, h1_ref, st_ref, acc):
    g = pl.program_id(0)
    xj = jnp.reshape(xj_ref[...], (_G * _K, _CIN))
    a = jnp.dot(xj, w1x_ref[...], preferred_element_type=jnp.float32)
    npw = jnp.dot(np_ref[...], w1p_ref[...], preferred_element_type=jnp.float32)
    h1 = jnp.reshape(a, (_G, _K, _CMID)) + pwj_ref[...] - npw[:, None, :]
    h1_ref[...] = h1
    h1f = jnp.reshape(h1, (_G * _K, _CMID))
    s = jnp.sum(h1f, axis=0, keepdims=True)
    sq = jnp.sum(h1f * h1f, axis=0, keepdims=True)

    @pl.when(g == 0)
    def _():
        acc[...] = jnp.zeros_like(acc)

    acc[0:1, :] += s
    acc[1:2, :] += sq

    @pl.when(g == _NG - 1)
    def _():
        st_ref[...] = acc[...]


def _conv1(xj3, pwj3, npad, w1x_t, w1p_t):
    return pl.pallas_call(
        _c1_body,
        grid=(_NG,),
        in_specs=[
            pl.BlockSpec((_G, _K, _CIN), lambda g: (g, 0, 0)),
            pl.BlockSpec((_G, _K, _CMID), lambda g: (g, 0, 0)),
            pl.BlockSpec((_G, _GD), lambda g: (g, 0)),
            pl.BlockSpec((_CIN, _CMID), lambda g: (0, 0)),
            pl.BlockSpec((_GD, _CMID), lambda g: (0, 0)),
        ],
        out_specs=(
            pl.BlockSpec((_G, _K, _CMID), lambda g: (g, 0, 0)),
            pl.BlockSpec((8, _CMID), lambda g: (0, 0)),
        ),
        out_shape=(
            jax.ShapeDtypeStruct((_BM, _K, _CMID), jnp.float32),
            jax.ShapeDtypeStruct((8, _CMID), jnp.float32),
        ),
        scratch_shapes=[pltpu.VMEM((8, _CMID), jnp.float32)],
    )(xj3, pwj3, npad, w1x_t, w1p_t)


def _c2_body(h1_ref, st1_ref, w2_ref, g1_ref, b1_ref, h2_ref, st_ref, acc):
    g = pl.program_id(0)
    st = st1_ref[...]
    mean = st[0:1, :] / _CNT
    var = st[1:2, :] / _CNT - mean * mean
    inv = g1_ref[...] / jnp.sqrt(var + _EPS)
    h1 = jnp.reshape(h1_ref[...], (_G * _K, _CMID))
    h1n = jnp.maximum((h1 - mean) * inv + b1_ref[...], 0.0)
    h2 = jnp.dot(h1n, w2_ref[...], preferred_element_type=jnp.float32)
    h2_ref[...] = jnp.reshape(h2, (_G, _K, _COUT))
    s = jnp.sum(h2, axis=0, keepdims=True)
    sq = jnp.sum(h2 * h2, axis=0, keepdims=True)

    @pl.when(g == 0)
    def _():
        acc[...] = jnp.zeros_like(acc)

    acc[0:1, :] += s
    acc[1:2, :] += sq

    @pl.when(g == _NG - 1)
    def _():
        st_ref[...] = acc[...]


def _conv2(h1, st1, w2_t, g1r, b1r):
    return pl.pallas_call(
        _c2_body,
        grid=(_NG,),
        in_specs=[
            pl.BlockSpec((_G, _K, _CMID), lambda g: (g, 0, 0)),
            pl.BlockSpec((8, _CMID), lambda g: (0, 0)),
            pl.BlockSpec((_CMID, _COUT), lambda g: (0, 0)),
            pl.BlockSpec((1, _CMID), lambda g: (0, 0)),
            pl.BlockSpec((1, _CMID), lambda g: (0, 0)),
        ],
        out_specs=(
            pl.BlockSpec((_G, _K, _COUT), lambda g: (g, 0, 0)),
            pl.BlockSpec((8, _COUT), lambda g: (0, 0)),
        ),
        out_shape=(
            jax.ShapeDtypeStruct((_BM, _K, _COUT), jnp.float32),
            jax.ShapeDtypeStruct((8, _COUT), jnp.float32),
        ),
        scratch_shapes=[pltpu.VMEM((8, _COUT), jnp.float32)],
    )(h1, st1, w2_t, g1r, b1r)


def _fin_body(h2_ref, st2_ref, id_ref, ws_ref, g2_ref, b2_ref, bs_ref, out_ref):
    st = st2_ref[...]
    mean = st[0:1, :] / _CNT
    var = st[1:2, :] / _CNT - mean * mean
    inv = g2_ref[...] / jnp.sqrt(var + _EPS)
    h2 = h2_ref[...]
    h2n = (h2 - mean[None, :, :]) * inv[None, :, :] + b2_ref[...][None, :, :]
    pooled = jnp.max(h2n, axis=1)
    skip = jnp.dot(id_ref[...], ws_ref[...], preferred_element_type=jnp.float32)
    out_ref[...] = jnp.maximum(pooled + skip + bs_ref[...], 0.0)


def _finalize(h2, st2, ident, ws_t, g2r, b2r, bsr):
    return pl.pallas_call(
        _fin_body,
        grid=(_NG,),
        in_specs=[
            pl.BlockSpec((_G, _K, _COUT), lambda g: (g, 0, 0)),
            pl.BlockSpec((8, _COUT), lambda g: (0, 0)),
            pl.BlockSpec((_G, _CIN), lambda g: (g, 0)),
            pl.BlockSpec((_CIN, _COUT), lambda g: (0, 0)),
            pl.BlockSpec((1, _COUT), lambda g: (0, 0)),
            pl.BlockSpec((1, _COUT), lambda g: (0, 0)),
            pl.BlockSpec((1, _COUT), lambda g: (0, 0)),
        ],
        out_specs=pl.BlockSpec((_G, _COUT), lambda g: (g, 0)),
        out_shape=jax.ShapeDtypeStruct((_BM, _COUT), jnp.float32),
    )(h2, st2, ident, ws_t, g2r, b2r, bsr)


def kernel(p, x, W1, g1, b1, W2, g2, b2, Wskip, bskip):
    px = p[:, :, 0]
    py = p[:, :, 1]
    pz = p[:, :, 2]
    idx_off, nx, ny, nz = _fps(px, py, pz)
    nbr_off = _ball_query(px, py, pz, nx, ny, nz)

    xt = jnp.reshape(jnp.transpose(x, (0, 2, 1)), (_B * _N, _CIN))
    ppad = jnp.concatenate(
        [jnp.reshape(p, (_B * _N, 3)),
         jnp.zeros((_B * _N, _GD - 3), jnp.float32)], axis=1)
    w1x_t = jnp.transpose(W1[:, 3:])
    w1p_t = jnp.concatenate(
        [jnp.transpose(W1[:, :3]),
         jnp.zeros((_GD - 3, _CMID), jnp.float32)], axis=0)
    pw = _pw(ppad, w1p_t)
    xj, pwj, ident = _sc_gather(
        xt, pw, jnp.reshape(nbr_off, (_S,)), jnp.reshape(idx_off, (_BM,)))

    new_p = jnp.stack([nx, ny, nz], axis=-1)
    npad = jnp.concatenate(
        [jnp.reshape(new_p, (_BM, 3)),
         jnp.zeros((_BM, _GD - 3), jnp.float32)], axis=1)

    h1, st1 = _conv1(jnp.reshape(xj, (_BM, _K, _CIN)),
                     jnp.reshape(pwj, (_BM, _K, _CMID)), npad, w1x_t, w1p_t)
    h2, st2 = _conv2(h1, st1, jnp.transpose(W2),
                     jnp.reshape(g1, (1, _CMID)), jnp.reshape(b1, (1, _CMID)))
    out_t = _finalize(h2, st2, ident, jnp.transpose(Wskip),
                      jnp.reshape(g2, (1, _COUT)), jnp.reshape(b2, (1, _COUT)),
                      jnp.reshape(bskip, (1, _COUT)))
    out = jnp.transpose(jnp.reshape(out_t, (_B, _M, _COUT)), (0, 2, 1))
    return new_p, out
